# Initial kernel scaffold; baseline (speedup 1.0000x reference)
import functools

import jax
import jax.numpy as jnp
from jax import lax
from jax.experimental import pallas as pl
from jax.experimental.pallas import tpu as pltpu
from jax.experimental.pallas import tpu_sc as plsc

NC, NS, L = 2, 16, 16
NW = NC * NS
S = 200
D = 32
CB = 4
G0 = 128
G1 = S - G0


def _sc_pool(inputs, table):
  B = inputs.shape[0]
  bw = B // NW
  nit = bw // CB
  mesh = plsc.VectorSubcoreMesh(core_axis_name="c", subcore_axis_name="s")

  @functools.partial(
      pl.kernel,
      out_type=jax.ShapeDtypeStruct((B, D), jnp.float32),
      mesh=mesh,
      scratch_types=[
          pltpu.VMEM((CB, S), jnp.int32),
          pltpu.VMEM((CB, S), jnp.int32),
          pltpu.VMEM((CB * S, D), jnp.float32),
          pltpu.VMEM((CB * S, D), jnp.float32),
          pltpu.VMEM((CB, D), jnp.float32),
          pltpu.VMEM((1, D), jnp.float32),
          pltpu.SemaphoreType.DMA,
          pltpu.SemaphoreType.DMA,
      ],
  )
  def k(inputs_hbm, table_hbm, out_hbm, idx0, idx1, rows0, rows1, outb, t0,
        sem0, sem1):
    wid = lax.axis_index("s") * NC + lax.axis_index("c")
    base = wid * bw

    pltpu.sync_copy(table_hbm.at[pl.ds(0, 1)], t0)

    def gather_copies(idx_ref, rows_ref, sem):
      copies = []
      for i in range(CB):
        copies.append(pltpu.make_async_copy(
            table_hbm.at[idx_ref.at[i, pl.ds(0, G0)]],
            rows_ref.at[pl.ds(i * S, G0)], sem))
        copies.append(pltpu.make_async_copy(
            table_hbm.at[idx_ref.at[i, pl.ds(G0, G1)]],
            rows_ref.at[pl.ds(i * S + G0, G1)], sem))
      return copies

    def fire(g, idx_ref, rows_ref, sem):
      pltpu.sync_copy(inputs_hbm.at[pl.ds(base + g * CB, CB)], idx_ref)
      for c in gather_copies(idx_ref, rows_ref, sem):
        c.start()

    def drain(idx_ref, rows_ref, sem):
      for c in gather_copies(idx_ref, rows_ref, sem):
        c.wait()

    def compute(g, idx_ref, rows_ref):
      for i in range(CB):
        zc = jnp.zeros((L,), jnp.int32)
        one = jnp.ones((L,), jnp.int32)
        zero = jnp.zeros((L,), jnp.int32)
        for kv in range(S // L):
          v = idx_ref[i, pl.ds(kv * L, L)]
          zc = zc + jnp.where(v == 0, one, zero)
        v = idx_ref[i, pl.ds(S - L, L)]
        lanes = lax.iota(jnp.int32, L)
        tail_new = L - (S - (S // L) * L)
        zc = zc + jnp.where((lanes >= tail_new) & (v == 0), one, zero)
        n0 = jnp.sum(zc)

        rbase = i * S
        z = jnp.zeros((L,), jnp.float32)

        def sbody(kk, acc):
          a0, b0, a1, b1 = acc
          r0 = rbase + 2 * kk
          a0 = a0 + rows_ref[r0, pl.ds(0, L)]
          b0 = b0 + rows_ref[r0, pl.ds(L, L)]
          a1 = a1 + rows_ref[r0 + 1, pl.ds(0, L)]
          b1 = b1 + rows_ref[r0 + 1, pl.ds(L, L)]
          return a0, b0, a1, b1

        a0, b0, a1, b1 = lax.fori_loop(0, S // 2, sbody, (z, z, z, z))
        sa = a0 + a1
        sb = b0 + b1

        n0f = n0.astype(jnp.float32)
        den = jnp.maximum(jnp.float32(S) - n0f, jnp.float32(1.0))
        ta = t0[0, pl.ds(0, L)]
        tb = t0[0, pl.ds(L, L)]
        outb[i, pl.ds(0, L)] = (sa - n0f * ta) / den
        outb[i, pl.ds(L, L)] = (sb - n0f * tb) / den
      pltpu.sync_copy(outb, out_hbm.at[pl.ds(base + g * CB, CB)])

    fire(0, idx0, rows0, sem0)

    def body2(g2, carry):
      g = g2 * 2
      fire(g + 1, idx1, rows1, sem1)
      drain(idx0, rows0, sem0)
      compute(g, idx0, rows0)

      @pl.when(g2 + 1 < nit // 2)
      def _():
        fire(g + 2, idx0, rows0, sem0)

      drain(idx1, rows1, sem1)
      compute(g + 1, idx1, rows1)
      return carry

    lax.fori_loop(0, nit // 2, body2, 0)

  return k(inputs, table)


def kernel(inputs, table):
  return _sc_pool(inputs, table)


# speedup vs baseline: 14.4678x; 14.4678x over previous
import functools

import jax
import jax.numpy as jnp
from jax import lax
from jax.experimental import pallas as pl
from jax.experimental.pallas import tpu as pltpu
from jax.experimental.pallas import tpu_sc as plsc

NC, NS, L = 2, 16, 16
NW = NC * NS
S = 200
D = 32
CB = 4
G0 = 128
G1 = S - G0


def _sc_pool(inputs, table):
  B = inputs.shape[0]
  bw = B // NW
  nit = bw // CB
  mesh = plsc.VectorSubcoreMesh(core_axis_name="c", subcore_axis_name="s")

  @functools.partial(
      pl.kernel,
      out_type=jax.ShapeDtypeStruct((B, D), jnp.float32),
      mesh=mesh,
      compiler_params=pltpu.CompilerParams(
          needs_layout_passes=False, use_tc_tiling_on_sc=False),
      scratch_types=[
          pltpu.VMEM((CB, S), jnp.int32),
          pltpu.VMEM((CB, S), jnp.int32),
          pltpu.VMEM((CB * S, D), jnp.float32),
          pltpu.VMEM((CB * S, D), jnp.float32),
          pltpu.VMEM((CB, D), jnp.float32),
          pltpu.VMEM((1, D), jnp.float32),
          pltpu.SemaphoreType.DMA,
          pltpu.SemaphoreType.DMA,
      ],
  )
  def k(inputs_hbm, table_hbm, out_hbm, idx0, idx1, rows0, rows1, outb, t0,
        sem0, sem1):
    wid = lax.axis_index("s") * NC + lax.axis_index("c")
    base = wid * bw

    pltpu.sync_copy(table_hbm.at[pl.ds(0, 1)], t0)

    def gather_copies(idx_ref, rows_ref, sem):
      copies = []
      for i in range(CB):
        copies.append(pltpu.make_async_copy(
            table_hbm.at[idx_ref.at[i, pl.ds(0, G0)]],
            rows_ref.at[pl.ds(i * S, G0)], sem))
        copies.append(pltpu.make_async_copy(
            table_hbm.at[idx_ref.at[i, pl.ds(G0, G1)]],
            rows_ref.at[pl.ds(i * S + G0, G1)], sem))
      return copies

    def fire(g, idx_ref, rows_ref, sem):
      pltpu.sync_copy(inputs_hbm.at[pl.ds(base + g * CB, CB)], idx_ref)
      for c in gather_copies(idx_ref, rows_ref, sem):
        c.start()

    def drain(idx_ref, rows_ref, sem):
      for c in gather_copies(idx_ref, rows_ref, sem):
        c.wait()

    def compute(g, idx_ref, rows_ref):
      for i in range(CB):
        n0v = jnp.zeros((L,), jnp.int32)
        for kv in range(S // L):
          v = idx_ref[i, pl.ds(kv * L, L)]
          n0v = n0v + plsc.all_reduce_population_count(v == 0)
        v = idx_ref[i, pl.ds(S - L, L)]
        lanes = lax.iota(jnp.int32, L)
        tail_new = L - (S - (S // L) * L)
        n0v = n0v + plsc.all_reduce_population_count(
            (lanes >= tail_new) & (v == 0))

        rbase = i * S
        z = jnp.zeros((L,), jnp.float32)

        def sbody(kk, acc):
          a0, b0, a1, b1 = acc
          r0 = rbase + 2 * kk
          a0 = a0 + rows_ref[r0, pl.ds(0, L)]
          b0 = b0 + rows_ref[r0, pl.ds(L, L)]
          a1 = a1 + rows_ref[r0 + 1, pl.ds(0, L)]
          b1 = b1 + rows_ref[r0 + 1, pl.ds(L, L)]
          return a0, b0, a1, b1

        a0, b0, a1, b1 = lax.fori_loop(0, S // 2, sbody, (z, z, z, z))
        sa = a0 + a1
        sb = b0 + b1

        n0f = n0v.astype(jnp.float32)
        den = jnp.maximum(jnp.float32(S) - n0f, jnp.float32(1.0))
        ta = t0[0, pl.ds(0, L)]
        tb = t0[0, pl.ds(L, L)]
        outb[i, pl.ds(0, L)] = (sa - n0f * ta) / den
        outb[i, pl.ds(L, L)] = (sb - n0f * tb) / den
      pltpu.sync_copy(outb, out_hbm.at[pl.ds(base + g * CB, CB)])

    fire(0, idx0, rows0, sem0)

    def body2(g2, carry):
      g = g2 * 2
      fire(g + 1, idx1, rows1, sem1)
      drain(idx0, rows0, sem0)
      compute(g, idx0, rows0)

      @pl.when(g2 + 1 < nit // 2)
      def _():
        fire(g + 2, idx0, rows0, sem0)

      drain(idx1, rows1, sem1)
      compute(g + 1, idx1, rows1)
      return carry

    lax.fori_loop(0, nit // 2, body2, 0)

  return k(inputs, table)


def kernel(inputs, table):
  return _sc_pool(inputs, table)


# speedup vs baseline: 15.0909x; 1.0431x over previous
import functools

import jax
import jax.numpy as jnp
from jax import lax
from jax.experimental import pallas as pl
from jax.experimental.pallas import tpu as pltpu
from jax.experimental.pallas import tpu_sc as plsc

NC, NS, L = 2, 16, 16
NW = NC * NS
S = 200
D = 32
CB = 8
G0 = 128
G1 = S - G0


def _sc_pool(inputs, table):
  B = inputs.shape[0]
  bw = B // NW
  nit = bw // CB
  mesh = plsc.VectorSubcoreMesh(core_axis_name="c", subcore_axis_name="s")

  @functools.partial(
      pl.kernel,
      out_type=jax.ShapeDtypeStruct((B, D), jnp.float32),
      mesh=mesh,
      compiler_params=pltpu.CompilerParams(
          needs_layout_passes=False, use_tc_tiling_on_sc=False),
      scratch_types=[
          pltpu.VMEM((CB, S), jnp.int32),
          pltpu.VMEM((CB, S), jnp.int32),
          pltpu.VMEM((CB * S, D), jnp.float32),
          pltpu.VMEM((CB * S, D), jnp.float32),
          pltpu.VMEM((CB, D), jnp.float32),
          pltpu.VMEM((1, D), jnp.float32),
          pltpu.SemaphoreType.DMA,
          pltpu.SemaphoreType.DMA,
      ],
  )
  def k(inputs_hbm, table_hbm, out_hbm, idx0, idx1, rows0, rows1, outb, t0,
        sem0, sem1):
    wid = lax.axis_index("s") * NC + lax.axis_index("c")
    base = wid * bw

    pltpu.sync_copy(table_hbm.at[pl.ds(0, 1)], t0)

    def gather_copies(idx_ref, rows_ref, sem):
      copies = []
      for i in range(CB):
        copies.append(pltpu.make_async_copy(
            table_hbm.at[idx_ref.at[i, pl.ds(0, G0)]],
            rows_ref.at[pl.ds(i * S, G0)], sem))
        copies.append(pltpu.make_async_copy(
            table_hbm.at[idx_ref.at[i, pl.ds(G0, G1)]],
            rows_ref.at[pl.ds(i * S + G0, G1)], sem))
      return copies

    def fire(g, idx_ref, rows_ref, sem):
      pltpu.sync_copy(inputs_hbm.at[pl.ds(base + g * CB, CB)], idx_ref)
      for c in gather_copies(idx_ref, rows_ref, sem):
        c.start()

    def drain(idx_ref, rows_ref, sem):
      for c in gather_copies(idx_ref, rows_ref, sem):
        c.wait()

    def compute(g, idx_ref, rows_ref):
      for i in range(CB):
        n0v = jnp.zeros((L,), jnp.int32)
        for kv in range(S // L):
          v = idx_ref[i, pl.ds(kv * L, L)]
          n0v = n0v + plsc.all_reduce_population_count(v == 0)
        v = idx_ref[i, pl.ds(S - L, L)]
        lanes = lax.iota(jnp.int32, L)
        tail_new = L - (S - (S // L) * L)
        n0v = n0v + plsc.all_reduce_population_count(
            (lanes >= tail_new) & (v == 0))

        rbase = i * S
        z = jnp.zeros((L,), jnp.float32)

        def sbody(kk, acc):
          a0, b0, a1, b1 = acc
          r0 = rbase + 2 * kk
          a0 = a0 + rows_ref[r0, pl.ds(0, L)]
          b0 = b0 + rows_ref[r0, pl.ds(L, L)]
          a1 = a1 + rows_ref[r0 + 1, pl.ds(0, L)]
          b1 = b1 + rows_ref[r0 + 1, pl.ds(L, L)]
          return a0, b0, a1, b1

        a0, b0, a1, b1 = lax.fori_loop(0, S // 2, sbody, (z, z, z, z))
        sa = a0 + a1
        sb = b0 + b1

        n0f = n0v.astype(jnp.float32)
        den = jnp.maximum(jnp.float32(S) - n0f, jnp.float32(1.0))
        ta = t0[0, pl.ds(0, L)]
        tb = t0[0, pl.ds(L, L)]
        outb[i, pl.ds(0, L)] = (sa - n0f * ta) / den
        outb[i, pl.ds(L, L)] = (sb - n0f * tb) / den
      pltpu.sync_copy(outb, out_hbm.at[pl.ds(base + g * CB, CB)])

    fire(0, idx0, rows0, sem0)

    def body2(g2, carry):
      g = g2 * 2
      fire(g + 1, idx1, rows1, sem1)
      drain(idx0, rows0, sem0)
      compute(g, idx0, rows0)

      @pl.when(g2 + 1 < nit // 2)
      def _():
        fire(g + 2, idx0, rows0, sem0)

      drain(idx1, rows1, sem1)
      compute(g + 1, idx1, rows1)
      return carry

    lax.fori_loop(0, nit // 2, body2, 0)

  return k(inputs, table)


def kernel(inputs, table):
  return _sc_pool(inputs, table)


# speedup vs baseline: 15.1734x; 1.0055x over previous
import functools

import jax
import jax.numpy as jnp
from jax import lax
from jax.experimental import pallas as pl
from jax.experimental.pallas import tpu as pltpu
from jax.experimental.pallas import tpu_sc as plsc

NC, NS, L = 2, 16, 16
NW = NC * NS
S = 200
D = 32
CB = 8
FLAT = CB * S
GCH = 128
NFULL = FLAT // GCH
TAIL = FLAT - NFULL * GCH


def _sc_pool(inputs, table):
  B = inputs.shape[0]
  bw = B // NW
  nit = bw // CB
  inputs_flat = inputs.reshape(-1)
  mesh = plsc.VectorSubcoreMesh(core_axis_name="c", subcore_axis_name="s")

  @functools.partial(
      pl.kernel,
      out_type=jax.ShapeDtypeStruct((B, D), jnp.float32),
      mesh=mesh,
      compiler_params=pltpu.CompilerParams(
          needs_layout_passes=False, use_tc_tiling_on_sc=False),
      scratch_types=[
          pltpu.VMEM((FLAT,), jnp.int32),
          pltpu.VMEM((FLAT,), jnp.int32),
          pltpu.VMEM((FLAT, D), jnp.float32),
          pltpu.VMEM((FLAT, D), jnp.float32),
          pltpu.VMEM((CB, D), jnp.float32),
          pltpu.VMEM((CB, D), jnp.float32),
          pltpu.VMEM((1, D), jnp.float32),
          pltpu.SemaphoreType.DMA,
          pltpu.SemaphoreType.DMA,
          pltpu.SemaphoreType.DMA,
          pltpu.SemaphoreType.DMA,
          pltpu.SemaphoreType.DMA,
          pltpu.SemaphoreType.DMA,
      ],
  )
  def k(inputs_hbm, table_hbm, out_hbm, idx0, idx1, rows0, rows1, os0, os1,
        t0, si0, si1, sr0, sr1, so0, so1):
    wid = lax.axis_index("s") * NC + lax.axis_index("c")
    base = wid * bw
    fbase = base * S

    idxs = (idx0, idx1)
    isems = (si0, si1)
    rows = (rows0, rows1)
    rsems = (sr0, sr1)
    outs = (os0, os1)
    osems = (so0, so1)

    pltpu.sync_copy(table_hbm.at[pl.ds(0, 1)], t0)

    def idx_copy(g, b):
      return pltpu.make_async_copy(
          inputs_hbm.at[pl.ds(fbase + g * FLAT, FLAT)], idxs[b], isems[b])

    def gather_copies(b):
      copies = []
      for c in range(NFULL):
        copies.append(pltpu.make_async_copy(
            table_hbm.at[idxs[b].at[pl.ds(c * GCH, GCH)]],
            rows[b].at[pl.ds(c * GCH, GCH)], rsems[b]))
      copies.append(pltpu.make_async_copy(
          table_hbm.at[idxs[b].at[pl.ds(NFULL * GCH, TAIL)]],
          rows[b].at[pl.ds(NFULL * GCH, TAIL)], rsems[b]))
      return copies

    def out_copy(g, b):
      return pltpu.make_async_copy(
          outs[b], out_hbm.at[pl.ds(base + g * CB, CB)], osems[b])

    def compute(idx_ref, rows_ref, outb):
      for i in range(CB):
        ib = i * S
        n0v = jnp.zeros((L,), jnp.int32)
        for kv in range(S // L):
          v = idx_ref[pl.ds(ib + kv * L, L)]
          n0v = n0v + plsc.all_reduce_population_count(v == 0)
        v = idx_ref[pl.ds(ib + S - L, L)]
        lanes = lax.iota(jnp.int32, L)
        tail_new = L - (S - (S // L) * L)
        n0v = n0v + plsc.all_reduce_population_count(
            (lanes >= tail_new) & (v == 0))

        z = jnp.zeros((L,), jnp.float32)

        def sbody(kk, acc):
          a0, b0, a1, b1 = acc
          r0 = ib + 2 * kk
          a0 = a0 + rows_ref[r0, pl.ds(0, L)]
          b0 = b0 + rows_ref[r0, pl.ds(L, L)]
          a1 = a1 + rows_ref[r0 + 1, pl.ds(0, L)]
          b1 = b1 + rows_ref[r0 + 1, pl.ds(L, L)]
          return a0, b0, a1, b1

        a0, b0, a1, b1 = lax.fori_loop(0, S // 2, sbody, (z, z, z, z))
        sa = a0 + a1
        sb = b0 + b1

        n0f = n0v.astype(jnp.float32)
        den = jnp.maximum(jnp.float32(S) - n0f, jnp.float32(1.0))
        ta = t0[0, pl.ds(0, L)]
        tb = t0[0, pl.ds(L, L)]
        outb[i, pl.ds(0, L)] = (sa - n0f * ta) / den
        outb[i, pl.ds(L, L)] = (sb - n0f * tb) / den

    idx_copy(0, 0).start()
    idx_copy(1, 1).start()
    idx_copy(0, 0).wait()
    for c in gather_copies(0):
      c.start()
    idx_copy(1, 1).wait()
    for c in gather_copies(1):
      c.start()

    def body2(g2, carry):
      for b in range(2):
        g = g2 * 2 + b
        for c in gather_copies(b):
          c.wait()
        @pl.when(g2 >= 1)
        def _():
          out_copy(g - 2, b).wait()
        compute(idxs[b], rows[b], outs[b])
        out_copy(g, b).start()
        @pl.when(g + 2 < nit)
        def _():
          idx_copy(g + 2, b).start()
          idx_copy(g + 2, b).wait()
          for c in gather_copies(b):
            c.start()
      return carry

    lax.fori_loop(0, nit // 2, body2, 0)

    out_copy(nit - 2, 0).wait()
    out_copy(nit - 1, 1).wait()

  return k(inputs_flat, table)


def kernel(inputs, table):
  return _sc_pool(inputs, table)


# speedup vs baseline: 16.3618x; 1.0783x over previous
import functools

import jax
import jax.numpy as jnp
from jax import lax
from jax.experimental import pallas as pl
from jax.experimental.pallas import tpu as pltpu
from jax.experimental.pallas import tpu_sc as plsc

NC, NS, L = 2, 16, 16
NW = NC * NS
S = 200
D = 32
CB = 8
FLAT = CB * S
GCH = 128
NFULL = FLAT // GCH
TAIL = FLAT - NFULL * GCH


def _sc_pool(inputs, table):
  B = inputs.shape[0]
  bw = B // NW
  nit = bw // CB
  inputs_flat = inputs.reshape(-1)
  mesh = plsc.VectorSubcoreMesh(core_axis_name="c", subcore_axis_name="s")

  @functools.partial(
      pl.kernel,
      out_type=jax.ShapeDtypeStruct((B, D), jnp.float32),
      mesh=mesh,
      compiler_params=pltpu.CompilerParams(
          needs_layout_passes=False, use_tc_tiling_on_sc=False),
      scratch_types=[
          pltpu.VMEM((FLAT,), jnp.int32),
          pltpu.VMEM((FLAT,), jnp.int32),
          pltpu.VMEM((FLAT, D), jnp.float32),
          pltpu.VMEM((FLAT, D), jnp.float32),
          pltpu.VMEM((CB, D), jnp.float32),
          pltpu.VMEM((CB, D), jnp.float32),
          pltpu.VMEM((1, D), jnp.float32),
          pltpu.SemaphoreType.DMA,
          pltpu.SemaphoreType.DMA,
          pltpu.SemaphoreType.DMA,
          pltpu.SemaphoreType.DMA,
          pltpu.SemaphoreType.DMA,
          pltpu.SemaphoreType.DMA,
      ],
  )
  def k(inputs_hbm, table_hbm, out_hbm, idx0, idx1, rows0, rows1, os0, os1,
        t0, si0, si1, sr0, sr1, so0, so1):
    wid = lax.axis_index("s") * NC + lax.axis_index("c")
    base = wid * bw
    fbase = base * S

    idxs = (idx0, idx1)
    isems = (si0, si1)
    rows = (rows0, rows1)
    rsems = (sr0, sr1)
    outs = (os0, os1)
    osems = (so0, so1)

    pltpu.sync_copy(table_hbm.at[pl.ds(0, 1)], t0)

    def idx_copy(g, b):
      return pltpu.make_async_copy(
          inputs_hbm.at[pl.ds(fbase + g * FLAT, FLAT)], idxs[b], isems[b])

    def gather_copies(b):
      copies = []
      for c in range(NFULL):
        copies.append(pltpu.make_async_copy(
            table_hbm.at[idxs[b].at[pl.ds(c * GCH, GCH)]],
            rows[b].at[pl.ds(c * GCH, GCH)], rsems[b]))
      copies.append(pltpu.make_async_copy(
          table_hbm.at[idxs[b].at[pl.ds(NFULL * GCH, TAIL)]],
          rows[b].at[pl.ds(NFULL * GCH, TAIL)], rsems[b]))
      return copies

    def out_copy(g, b):
      return pltpu.make_async_copy(
          outs[b], out_hbm.at[pl.ds(base + g * CB, CB)], osems[b])

    def compute(idx_ref, rows_ref, outb):
      for i in range(CB):
        ib = i * S
        n0v = jnp.zeros((L,), jnp.int32)
        for kv in range(S // L):
          v = idx_ref[pl.ds(ib + kv * L, L)]
          n0v = n0v + plsc.all_reduce_population_count(v == 0)
        v = idx_ref[pl.ds(ib + S - L, L)]
        lanes = lax.iota(jnp.int32, L)
        tail_new = L - (S - (S // L) * L)
        n0v = n0v + plsc.all_reduce_population_count(
            (lanes >= tail_new) & (v == 0))

        z = jnp.zeros((L,), jnp.float32)

        def sbody(kk, acc):
          accs = list(acc)
          r0 = ib + 4 * kk
          for u in range(4):
            accs[2 * u] = accs[2 * u] + rows_ref[r0 + u, pl.ds(0, L)]
            accs[2 * u + 1] = accs[2 * u + 1] + rows_ref[r0 + u, pl.ds(L, L)]
          return tuple(accs)

        acc = lax.fori_loop(0, S // 4, sbody, (z,) * 8)
        sa = (acc[0] + acc[2]) + (acc[4] + acc[6])
        sb = (acc[1] + acc[3]) + (acc[5] + acc[7])

        n0f = n0v.astype(jnp.float32)
        den = jnp.maximum(jnp.float32(S) - n0f, jnp.float32(1.0))
        ta = t0[0, pl.ds(0, L)]
        tb = t0[0, pl.ds(L, L)]
        outb[i, pl.ds(0, L)] = (sa - n0f * ta) / den
        outb[i, pl.ds(L, L)] = (sb - n0f * tb) / den

    idx_copy(0, 0).start()
    idx_copy(1, 1).start()
    idx_copy(0, 0).wait()
    for c in gather_copies(0):
      c.start()
    idx_copy(1, 1).wait()
    for c in gather_copies(1):
      c.start()

    def body2(g2, carry):
      for b in range(2):
        g = g2 * 2 + b
        for c in gather_copies(b):
          c.wait()
        @pl.when(g2 >= 1)
        def _():
          out_copy(g - 2, b).wait()
        compute(idxs[b], rows[b], outs[b])
        out_copy(g, b).start()
        @pl.when(g + 2 < nit)
        def _():
          idx_copy(g + 2, b).start()
          idx_copy(g + 2, b).wait()
          for c in gather_copies(b):
            c.start()
      return carry

    lax.fori_loop(0, nit // 2, body2, 0)

    out_copy(nit - 2, 0).wait()
    out_copy(nit - 1, 1).wait()

  return k(inputs_flat, table)


def kernel(inputs, table):
  return _sc_pool(inputs, table)
